# baseline (speedup 1.0000x reference)
import functools
import jax
import jax.numpy as jnp
from jax import lax
from jax.experimental import pallas as pl
from jax.experimental.pallas import tpu as pltpu
from jax.experimental.pallas import tpu_sc as plsc

N = 10000
E = 160000
H = 4
C = 128
DHC = H * C
CH = 128
OUT = 64
DIN = 256

BM = 256

NC = 2
NS = 16
NW = NC * NS
ROWS = 313
NPAD = NW * ROWS
FCH = 8000
NCHUNK = E // FCH
CAP = E + NCHUNK * 16 + 128 + 512
ECH = 128
ECHA = 512
ECHB = 64
DENW = 320


def _grid(n):
    return (n + BM - 1) // BM


def _s1_body(xp_ref, xd_ref, Wsp_ref, Wsd_ref, Asp_ref, Asd_ref,
             vdp_ref, vdd_ref,
             hsp_ref, esp_ref, edp_ref, hsd_ref, esd_ref, edd_ref):
    xp = xp_ref[...]
    xd = xd_ref[...]
    hs_p = jnp.dot(xp, Wsp_ref[...], preferred_element_type=jnp.float32)
    hs_d = jnp.dot(xd, Wsd_ref[...], preferred_element_type=jnp.float32)
    for p in range(2):
        hsp_ref[p] = hs_p[:, p * 2 * C:(p + 1) * 2 * C]
        hsd_ref[p] = hs_d[:, p * 2 * C:(p + 1) * 2 * C]
    esp_ref[...] = jnp.dot(hs_p, Asp_ref[...], preferred_element_type=jnp.float32)
    esd_ref[...] = jnp.dot(hs_d, Asd_ref[...], preferred_element_type=jnp.float32)
    edp_ref[...] = jnp.dot(xd, vdp_ref[...], preferred_element_type=jnp.float32)
    edd_ref[...] = jnp.dot(xp, vdd_ref[...], preferred_element_type=jnp.float32)


def _stage1(xp, xd, Wsp, Wsd, Asp, Asd, vdp, vdd):
    g = _grid(N)
    full = lambda shape: pl.BlockSpec(shape, lambda i: (0,) * len(shape))
    row2 = pl.BlockSpec((BM, DIN), lambda i: (i, 0))
    outs = (
        jax.ShapeDtypeStruct((2, N, 2 * C), jnp.float32),
        jax.ShapeDtypeStruct((N, H), jnp.float32),
        jax.ShapeDtypeStruct((N, H), jnp.float32),
        jax.ShapeDtypeStruct((2, N, 2 * C), jnp.float32),
        jax.ShapeDtypeStruct((N, H), jnp.float32),
        jax.ShapeDtypeStruct((N, H), jnp.float32),
    )
    hs_spec = pl.BlockSpec((2, BM, 2 * C), lambda i: (0, i, 0))
    sc_spec = pl.BlockSpec((BM, H), lambda i: (i, 0))
    return pl.pallas_call(
        _s1_body,
        grid=(g,),
        in_specs=[row2, row2, full((DIN, DHC)), full((DIN, DHC)),
                  full((DHC, H)), full((DHC, H)), full((DIN, H)), full((DIN, H))],
        out_specs=(hs_spec, sc_spec, sc_spec, hs_spec, sc_spec, sc_spec),
        out_shape=outs,
    )(xp, xd, Wsp, Wsd, Asp, Asd, vdp, vdd)


def _s5d_body(raw_ref, den_ref, b_ref, W_ref, lb_ref, W2_ref, as2_ref,
              hs2_ref, es2_ref):
    parts = []
    for h in range(H):
        den = den_ref[h][:, None] + 1e-16
        pr, sub = divmod(h, 2)
        parts.append(raw_ref[pr][:, sub * C:(sub + 1) * C] / den
                     + b_ref[pl.ds(h * C, C)][None, :])
    d1 = jnp.concatenate(parts, axis=1)
    y = jax.nn.relu(d1 + jnp.dot(d1, W_ref[...], preferred_element_type=jnp.float32)
                    + lb_ref[...][None, :])
    hs2 = jnp.dot(y, W2_ref[...], preferred_element_type=jnp.float32)
    hs2_ref[...] = hs2
    es2_ref[...] = jnp.dot(hs2, as2_ref[...], preferred_element_type=jnp.float32)


def _stage5d(raw, den, gb, W, lb, W2, as2):
    g = _grid(N)
    npad = raw.shape[1]
    full = lambda shape: pl.BlockSpec(shape, lambda i: (0,) * len(shape))
    return pl.pallas_call(
        _s5d_body,
        grid=(g,),
        in_specs=[pl.BlockSpec((2, BM, 2 * C), lambda i: (0, i, 0)),
                  pl.BlockSpec((H, BM), lambda i: (0, i)),
                  full((DHC,)), full((DHC, DHC)), full((DHC,)),
                  full((DHC, CH)), full((CH, 1))],
        out_specs=(pl.BlockSpec((BM, CH), lambda i: (i, 0)),
                   pl.BlockSpec((BM, 1), lambda i: (i, 0))),
        out_shape=(jax.ShapeDtypeStruct((N, CH), jnp.float32),
                   jax.ShapeDtypeStruct((N, 1), jnp.float32)),
    )(raw, den, gb, W, lb, W2, as2)


def _s5p_body(raw_ref, den_ref, b_ref, W_ref, lb_ref, v2_ref, ed2_ref):
    parts = []
    for h in range(H):
        den = den_ref[h][:, None] + 1e-16
        pr, sub = divmod(h, 2)
        parts.append(raw_ref[pr][:, sub * C:(sub + 1) * C] / den
                     + b_ref[pl.ds(h * C, C)][None, :])
    p1 = jnp.concatenate(parts, axis=1)
    y = jax.nn.relu(p1 + jnp.dot(p1, W_ref[...], preferred_element_type=jnp.float32)
                    + lb_ref[...][None, :])
    ed2_ref[...] = jnp.dot(y, v2_ref[...], preferred_element_type=jnp.float32)


def _stage5p(raw, den, gb, W, lb, v2):
    g = _grid(N)
    full = lambda shape: pl.BlockSpec(shape, lambda i: (0,) * len(shape))
    return pl.pallas_call(
        _s5p_body,
        grid=(g,),
        in_specs=[pl.BlockSpec((2, BM, 2 * C), lambda i: (0, i, 0)),
                  pl.BlockSpec((H, BM), lambda i: (0, i)),
                  full((DHC,)), full((DHC, DHC)), full((DHC,)), full((DHC, 1))],
        out_specs=pl.BlockSpec((BM, 1), lambda i: (i, 0)),
        out_shape=jax.ShapeDtypeStruct((N, 1), jnp.float32),
    )(raw, den, gb, W, lb, v2)


def _s8_body(raw_ref, den_ref, b2_ref, W2_ref, lb2_ref, W3_ref, lb3_ref, out_ref):
    p2 = raw_ref[...] / (den_ref[...] + 1e-16) + b2_ref[...][None, :]
    p2 = jax.nn.relu(p2 + jnp.dot(p2, W2_ref[...], preferred_element_type=jnp.float32)
                     + lb2_ref[...][None, :])
    lg = jnp.dot(p2, W3_ref[...], preferred_element_type=jnp.float32) + lb3_ref[...][None, :]
    m = jnp.max(lg, axis=1, keepdims=True)
    ex = jnp.exp(lg - m)
    out_ref[...] = lg - m - jnp.log(jnp.sum(ex, axis=1, keepdims=True))


def _stage8(raw2, den2, b2, W2, lb2, W3, lb3):
    g = _grid(N)
    full = lambda shape: pl.BlockSpec(shape, lambda i: (0,) * len(shape))
    return pl.pallas_call(
        _s8_body,
        grid=(g,),
        in_specs=[pl.BlockSpec((BM, CH), lambda i: (i, 0)),
                  pl.BlockSpec((BM, 1), lambda i: (i, 0)),
                  full((CH,)), full((CH, CH)), full((CH,)),
                  full((CH, OUT)), full((OUT,))],
        out_specs=pl.BlockSpec((BM, OUT), lambda i: (i, 0)),
        out_shape=jax.ShapeDtypeStruct((N, OUT), jnp.float32),
    )(raw2, den2, b2, W2, lb2, W3, lb3)


def _filter_sc(s, d):
    mesh = plsc.VectorSubcoreMesh(core_axis_name="c", subcore_axis_name="s")

    @functools.partial(
        pl.kernel,
        out_type=(jax.ShapeDtypeStruct((NW * CAP,), jnp.int32),
                  jax.ShapeDtypeStruct((NW * CAP,), jnp.int32),
                  jax.ShapeDtypeStruct((NW * 16,), jnp.int32)),
        mesh=mesh,
        compiler_params=pltpu.CompilerParams(needs_layout_passes=False),
        scratch_types=[pltpu.VMEM((FCH,), jnp.int32),
                       pltpu.VMEM((FCH,), jnp.int32),
                       pltpu.VMEM((FCH + 32,), jnp.int32),
                       pltpu.VMEM((FCH + 32,), jnp.int32)],
    )
    def filt(s_hbm, d_hbm, sb_hbm, db_hbm, cnt_hbm, s_in, d_in, st_s, st_d):
        wid = lax.axis_index("s") * NC + lax.axis_index("c")
        lo = wid * ROWS
        wbase = wid * CAP

        def chunk_body(ck, cursor):
            pltpu.sync_copy(s_hbm.at[pl.ds(ck * FCH, FCH)], s_in)
            pltpu.sync_copy(d_hbm.at[pl.ds(ck * FCH, FCH)], d_in)

            def vec_body(i, cnt):
                sv = s_in[pl.ds(i * 16, 16)]
                dv = d_in[pl.ds(i * 16, 16)]
                m = (dv >= lo) & (dv < lo + ROWS)
                mi = m.astype(jnp.int32)
                excl = plsc.cumsum(mi) - mi
                idx = jnp.where(m, cnt + excl, FCH + 16)
                plsc.store_scatter(st_s, [idx], sv)
                plsc.store_scatter(st_d, [idx], dv - lo)
                return cnt + jnp.sum(mi)

            cnt = lax.fori_loop(0, FCH // 16, vec_body, jnp.int32(0))
            st_s[pl.ds(cnt, 16)] = jnp.zeros((16,), jnp.int32)
            st_d[pl.ds(cnt, 16)] = jnp.full((16,), ROWS, jnp.int32)
            cnt_pad = ((cnt + 15) // 16) * 16
            off = pl.multiple_of(wbase + cursor, 16)
            pltpu.sync_copy(st_s.at[pl.ds(0, FCH + 16)],
                            sb_hbm.at[pl.ds(off, FCH + 16)])
            pltpu.sync_copy(st_d.at[pl.ds(0, FCH + 16)],
                            db_hbm.at[pl.ds(off, FCH + 16)])
            return cursor + cnt_pad

        cursor = lax.fori_loop(0, NCHUNK, chunk_body, jnp.int32(0))
        for j in range(ECH // 16):
            st_s[pl.ds(j * 16, 16)] = jnp.zeros((16,), jnp.int32)
            st_d[pl.ds(j * 16, 16)] = jnp.full((16,), ROWS, jnp.int32)
        off = pl.multiple_of(wbase + cursor, 16)
        pltpu.sync_copy(st_s.at[pl.ds(0, ECH)], sb_hbm.at[pl.ds(off, ECH)])
        pltpu.sync_copy(st_d.at[pl.ds(0, ECH)], db_hbm.at[pl.ds(off, ECH)])
        nch = (cursor + ECH - 1) // ECH
        st_s[pl.ds(0, 16)] = jnp.full((16,), 1, jnp.int32) * nch
        pltpu.sync_copy(st_s.at[pl.ds(0, 16)],
                        cnt_hbm.at[pl.ds(pl.multiple_of(wid * 16, 16), 16)])

    return filt(s, d)


def _gat_pass_sc(hs_list, es_T, ed_T, sb, db, cnt):
    nh = len(hs_list)
    mesh = plsc.VectorSubcoreMesh(core_axis_name="c", subcore_axis_name="s")

    @functools.partial(
        pl.kernel,
        out_type=(jax.ShapeDtypeStruct((nh * NPAD * C,), jnp.float32),
                  jax.ShapeDtypeStruct((nh * NW * DENW,), jnp.float32)),
        mesh=mesh,
        compiler_params=pltpu.CompilerParams(needs_layout_passes=False),
        scratch_types=[pltpu.VMEM(((ROWS + 1) * C,), jnp.float32),
                       pltpu.VMEM((ECH, C), jnp.float32),
                       pltpu.VMEM((ECH, C), jnp.float32),
                       pltpu.VMEM((N,), jnp.float32),
                       pltpu.VMEM((N,), jnp.float32),
                       pltpu.VMEM((DENW,), jnp.float32),
                       pltpu.VMEM((ECH,), jnp.int32),
                       pltpu.VMEM((ECH,), jnp.int32),
                       pltpu.VMEM((ECH,), jnp.int32),
                       pltpu.VMEM((ECH,), jnp.int32),
                       pltpu.VMEM((ECH,), jnp.float32),
                       pltpu.VMEM((16,), jnp.int32),
                       pltpu.SemaphoreType.DMA,
                       pltpu.SemaphoreType.DMA],
    )
    def gat(*refs):
        hs_refs = refs[:nh]
        es_hbm, ed_hbm, sb_hbm, db_hbm, cnt_hbm, raw_hbm, den_hbm = refs[nh:nh + 7]
        (out_f, rows0, rows1, es_v, ed_v, den_v, s_v0, s_v1, d_v0, d_v1,
         ex_v, cnt_v, sem0, sem1) = refs[nh + 7:]
        wid = lax.axis_index("s") * NC + lax.axis_index("c")
        lo = wid * ROWS
        wbase = wid * CAP
        pltpu.sync_copy(cnt_hbm.at[pl.ds(pl.multiple_of(wid * 16, 16), 16)],
                        cnt_v)
        nch = cnt_v[pl.ds(0, 16)][0]
        for h in range(nh):
            hs_ref = hs_refs[h]
            pltpu.sync_copy(es_hbm.at[pl.ds(h * N, N)], es_v)
            pltpu.sync_copy(ed_hbm.at[pl.ds(h * N, N)], ed_v)

            def zbody(i, _):
                out_f[pl.ds(i * 16, 16)] = jnp.zeros((16,), jnp.float32)
                return 0

            lax.fori_loop(0, (ROWS + 1) * C // 16, zbody, 0)
            for i in range(DENW // 16):
                den_v[pl.ds(i * 16, 16)] = jnp.zeros((16,), jnp.float32)

            def fetch(ck, s_v, d_v, rows, sem):
                @pl.when(ck < nch)
                def _():
                    eoff = pl.multiple_of(wbase + ck * ECH, 16)
                    pltpu.sync_copy(sb_hbm.at[pl.ds(eoff, ECH)], s_v)
                    pltpu.sync_copy(db_hbm.at[pl.ds(eoff, ECH)], d_v)
                    pltpu.make_async_copy(hs_ref.at[s_v], rows, sem).start()

            def process(ck, s_v, d_v, rows, sem):
                @pl.when(ck < nch)
                def _():
                    pltpu.make_async_copy(hs_ref.at[s_v], rows, sem).wait()

                    def vec_body(j, _):
                        sv = s_v[pl.ds(j * 16, 16)]
                        dv = d_v[pl.ds(j * 16, 16)]
                        esg = plsc.load_gather(es_v, [sv])
                        edi = jnp.minimum(dv + lo, N - 1)
                        edg = plsc.load_gather(ed_v, [edi])
                        e = esg + edg
                        e = jnp.where(e >= 0.0, e, 0.2 * e)
                        exv = jnp.exp(e)
                        plsc.addupdate_scatter(den_v, [dv], exv)
                        ex_v[pl.ds(j * 16, 16)] = exv
                        return 0

                    lax.fori_loop(0, ECH // 16, vec_body, 0)

                    def fma_body(k, _):
                        dv16 = d_v[pl.ds(k * 16, 16)]
                        ex16 = ex_v[pl.ds(k * 16, 16)]
                        for l in range(16):
                            base = dv16[l] * C
                            exb = jnp.full((16,), ex16[l])
                            for j in range(C // 16):
                                plsc.addupdate(
                                    out_f.at[pl.ds(base + j * 16, 16)],
                                    exb * rows[k * 16 + l, pl.ds(j * 16, 16)])
                        return 0

                    lax.fori_loop(0, ECH // 16, fma_body, 0)

            fetch(jnp.int32(0), s_v0, d_v0, rows0, sem0)

            def pair_body(k, _):
                c0 = k * 2
                fetch(c0 + 1, s_v1, d_v1, rows1, sem1)
                process(c0, s_v0, d_v0, rows0, sem0)
                fetch(c0 + 2, s_v0, d_v0, rows0, sem0)
                process(c0 + 1, s_v1, d_v1, rows1, sem1)
                return 0

            lax.fori_loop(0, (nch + 1) // 2, pair_body, 0)
            roff = pl.multiple_of(h * NPAD * C + lo * C, 16)
            pltpu.sync_copy(out_f.at[pl.ds(0, ROWS * C)],
                            raw_hbm.at[pl.ds(roff, ROWS * C)])
            doff = pl.multiple_of(h * NW * DENW + wid * DENW, 16)
            pltpu.sync_copy(den_v, den_hbm.at[pl.ds(doff, DENW)])

    raw_f, den_f = gat(*hs_list, es_T.reshape(-1), ed_T.reshape(-1), sb, db, cnt)
    raw = raw_f.reshape(nh, NPAD, C)
    den = den_f.reshape(nh, NW, DENW)[:, :, :ROWS].reshape(nh, NPAD)
    return raw, den


def _gat_passA_sc(es_i, ed_i, sb, db, cnt):
    mesh = plsc.VectorSubcoreMesh(core_axis_name="c", subcore_axis_name="s")

    @functools.partial(
        pl.kernel,
        out_type=(jax.ShapeDtypeStruct((NW * CAP * 2,), jnp.float32),
                  jax.ShapeDtypeStruct((NW * CAP * 2,), jnp.float32),
                  jax.ShapeDtypeStruct((NW * 4 * DENW,), jnp.float32)),
        mesh=mesh,
        compiler_params=pltpu.CompilerParams(needs_layout_passes=False),
        scratch_types=[pltpu.VMEM((4 * N,), jnp.float32),
                       pltpu.VMEM((4 * N,), jnp.float32),
                       pltpu.VMEM((ECHA,), jnp.int32),
                       pltpu.VMEM((ECHA,), jnp.int32),
                       pltpu.VMEM((2 * ECHA,), jnp.float32),
                       pltpu.VMEM((2 * ECHA,), jnp.float32),
                       pltpu.VMEM((4 * DENW,), jnp.float32),
                       pltpu.VMEM((16,), jnp.int32)],
    )
    def pA(es_hbm, ed_hbm, sb_hbm, db_hbm, cnt_hbm, ex0_hbm, ex1_hbm, den_hbm,
           es_v, ed_v, s_v, d_v, x0_v, x1_v, den_v, cnt_v):
        wid = lax.axis_index("s") * NC + lax.axis_index("c")
        lo = wid * ROWS
        wbase = wid * CAP
        pltpu.sync_copy(cnt_hbm.at[pl.ds(pl.multiple_of(wid * 16, 16), 16)],
                        cnt_v)
        nche = cnt_v[pl.ds(0, 16)][0] * ECH
        ncha = (nche + ECHA - 1) // ECHA
        pltpu.sync_copy(es_hbm, es_v)
        pltpu.sync_copy(ed_hbm, ed_v)
        for i in range(4 * DENW // 16):
            den_v[pl.ds(i * 16, 16)] = jnp.zeros((16,), jnp.float32)
        lane = lax.iota(jnp.int32, 16)

        def chunk_body(ck, _):
            eoff = pl.multiple_of(wbase + ck * ECHA, 16)
            pltpu.sync_copy(sb_hbm.at[pl.ds(eoff, ECHA)], s_v)
            pltpu.sync_copy(db_hbm.at[pl.ds(eoff, ECHA)], d_v)

            def vec_body(j, _):
                valid = (ck * ECHA + j * 16 + lane) < nche
                sv = jnp.where(valid, s_v[pl.ds(j * 16, 16)], 0)
                dv = jnp.where(valid, d_v[pl.ds(j * 16, 16)], ROWS)
                edi = jnp.minimum(dv + lo, N - 1) * 4
                svi = sv * 4
                for h in range(4):
                    esg = plsc.load_gather(es_v, [svi + h])
                    edg = plsc.load_gather(ed_v, [edi + h])
                    e = esg + edg
                    e = jnp.where(e >= 0.0, e, 0.2 * e)
                    exv = jnp.exp(e)
                    plsc.addupdate_scatter(den_v, [dv + h * DENW], exv)
                    xst = x0_v if h < 2 else x1_v
                    plsc.store_scatter(xst, [lane * 2 + (j * 32 + (h % 2))],
                                       exv)
                return 0

            lax.fori_loop(0, ECHA // 16, vec_body, 0)
            xoff = pl.multiple_of(2 * (wbase + ck * ECHA), 16)
            pltpu.sync_copy(x0_v, ex0_hbm.at[pl.ds(xoff, 2 * ECHA)])
            pltpu.sync_copy(x1_v, ex1_hbm.at[pl.ds(xoff, 2 * ECHA)])
            return 0

        lax.fori_loop(0, ncha, chunk_body, 0)
        doff = pl.multiple_of(wid * 4 * DENW, 16)
        pltpu.sync_copy(den_v, den_hbm.at[pl.ds(doff, 4 * DENW)])

    ex0, ex1, den_f = pA(es_i, ed_i, sb, db, cnt)
    den = (den_f.reshape(NW, 4, DENW)[:, :, :ROWS]
           .transpose(1, 0, 2).reshape(4, NPAD))
    return ex0, ex1, den


def _gat_passB_sc(hs_pairs, sb, db, cnt, ex0, ex1):
    mesh = plsc.VectorSubcoreMesh(core_axis_name="c", subcore_axis_name="s")
    C2 = 2 * C

    @functools.partial(
        pl.kernel,
        out_type=jax.ShapeDtypeStruct((2 * NPAD * C2,), jnp.float32),
        mesh=mesh,
        compiler_params=pltpu.CompilerParams(needs_layout_passes=False),
        scratch_types=[pltpu.VMEM(((ROWS + 1) * C2,), jnp.float32),
                       pltpu.VMEM((ECHB, C2), jnp.float32),
                       pltpu.VMEM((ECHB, C2), jnp.float32),
                       pltpu.VMEM((ECHB,), jnp.int32),
                       pltpu.VMEM((ECHB,), jnp.int32),
                       pltpu.VMEM((ECHB,), jnp.int32),
                       pltpu.VMEM((ECHB,), jnp.int32),
                       pltpu.VMEM((2 * ECHB,), jnp.float32),
                       pltpu.VMEM((2 * ECHB,), jnp.float32),
                       pltpu.VMEM((16,), jnp.int32),
                       pltpu.SemaphoreType.DMA,
                       pltpu.SemaphoreType.DMA],
    )
    def pB(hs0_hbm, hs1_hbm, sb_hbm, db_hbm, cnt_hbm, ex0_hbm, ex1_hbm,
           raw_hbm, out_f, rows0, rows1, s_v0, s_v1, d_v0, d_v1, x_v0, x_v1,
           cnt_v, sem0, sem1):
        wid = lax.axis_index("s") * NC + lax.axis_index("c")
        lo = wid * ROWS
        wbase = wid * CAP
        pltpu.sync_copy(cnt_hbm.at[pl.ds(pl.multiple_of(wid * 16, 16), 16)],
                        cnt_v)
        nchb = cnt_v[pl.ds(0, 16)][0] * (ECH // ECHB)
        for p in range(2):
            hs_ref = (hs0_hbm, hs1_hbm)[p]
            ex_ref = (ex0_hbm, ex1_hbm)[p]

            def zbody(i, _):
                out_f[pl.ds(i * 16, 16)] = jnp.zeros((16,), jnp.float32)
                return 0

            lax.fori_loop(0, (ROWS + 1) * C2 // 16, zbody, 0)

            def fetch(ck, s_v, d_v, x_v, rows, sem):
                @pl.when(ck < nchb)
                def _():
                    eoff = pl.multiple_of(wbase + ck * ECHB, 16)
                    pltpu.sync_copy(sb_hbm.at[pl.ds(eoff, ECHB)], s_v)
                    pltpu.sync_copy(db_hbm.at[pl.ds(eoff, ECHB)], d_v)
                    xoff = pl.multiple_of(2 * (wbase + ck * ECHB), 16)
                    pltpu.sync_copy(ex_ref.at[pl.ds(xoff, 2 * ECHB)], x_v)
                    pltpu.make_async_copy(hs_ref.at[s_v], rows, sem).start()

            def process(ck, s_v, d_v, x_v, rows, sem):
                @pl.when(ck < nchb)
                def _():
                    pltpu.make_async_copy(hs_ref.at[s_v], rows, sem).wait()

                    def fma_body(k, _):
                        dv16 = d_v[pl.ds(k * 16, 16)]
                        xa = x_v[pl.ds(k * 32, 16)]
                        xb = x_v[pl.ds(k * 32 + 16, 16)]
                        for l in range(16):
                            e = k * 16 + l
                            xv = xa if l < 8 else xb
                            base = dv16[l] * C2
                            for hh in range(2):
                                exb = jnp.full((16,), xv[(l % 8) * 2 + hh])
                                for j in range(C // 16):
                                    off = hh * C + j * 16
                                    plsc.addupdate(
                                        out_f.at[pl.ds(base + off, 16)],
                                        exb * rows[e, pl.ds(off, 16)])
                        return 0

                    lax.fori_loop(0, ECHB // 16, fma_body, 0)

            fetch(jnp.int32(0), s_v0, d_v0, x_v0, rows0, sem0)

            def pair_body(k, _):
                c0 = k * 2
                fetch(c0 + 1, s_v1, d_v1, x_v1, rows1, sem1)
                process(c0, s_v0, d_v0, x_v0, rows0, sem0)
                fetch(c0 + 2, s_v0, d_v0, x_v0, rows0, sem0)
                process(c0 + 1, s_v1, d_v1, x_v1, rows1, sem1)
                return 0

            lax.fori_loop(0, (nchb + 1) // 2, pair_body, 0)
            roff = pl.multiple_of(p * NPAD * C2 + lo * C2, 16)
            pltpu.sync_copy(out_f.at[pl.ds(0, ROWS * C2)],
                            raw_hbm.at[pl.ds(roff, ROWS * C2)])

    raw_f = pB(hs_pairs[0], hs_pairs[1], sb, db, cnt, ex0, ex1)
    return raw_f.reshape(2, NPAD, C2)


def _blockdiag_a(a):
    hh = a.shape[0]
    eye = jnp.eye(hh, dtype=a.dtype)
    A = eye[:, :, None] * a[:, None, :]
    return A.transpose(1, 2, 0).reshape(hh * a.shape[1], hh)


def kernel(x_person, x_diagnosis, edge_index_p2d, edge_index_d2p, edge_attr_p2d,
           g1p_Ws, g1p_Wd, g1p_as, g1p_ad, g1p_b,
           g1d_Ws, g1d_Wd, g1d_as, g1d_ad, g1d_b,
           g2p_W, g2p_as, g2p_ad, g2p_b,
           g2d_W, g2d_as, g2d_ad, g2d_b,
           lin1_W, lin1_b, lin2_W, lin2_b, lin3_W, lin3_b,
           em1_W, em1_b, em2_W, em2_b):
    Asp = _blockdiag_a(g1p_as)
    Adp = _blockdiag_a(g1p_ad)
    Asd = _blockdiag_a(g1d_as)
    Add = _blockdiag_a(g1d_ad)
    vdp = g1p_Wd @ Adp
    vdd = g1d_Wd @ Add
    as2 = g2d_as.reshape(CH, 1)
    vd2 = g2d_W @ g2d_ad.reshape(CH, 1)

    hs_p2d, es_p2d, ed_p2d, hs_d2p, es_d2p, ed_d2p = _stage1(
        x_person, x_diagnosis, g1p_Ws, g1d_Ws, Asp, Asd, vdp, vdd)

    s_p2d = edge_index_p2d[0]
    d_p2d = edge_index_p2d[1]
    s_d2p = edge_index_d2p[0]
    d_d2p = edge_index_d2p[1]

    sb_p, db_p, cnt_p = _filter_sc(s_p2d, d_p2d)
    sb_d, db_d, cnt_d = _filter_sc(s_d2p, d_d2p)

    ex0_p, ex1_p, den_d1 = _gat_passA_sc(
        es_p2d.reshape(-1), ed_p2d.reshape(-1), sb_p, db_p, cnt_p)
    raw_d1 = _gat_passB_sc(
        [hs_p2d[0], hs_p2d[1]], sb_p, db_p, cnt_p, ex0_p, ex1_p)
    ex0_d, ex1_d, den_p1 = _gat_passA_sc(
        es_d2p.reshape(-1), ed_d2p.reshape(-1), sb_d, db_d, cnt_d)
    raw_p1 = _gat_passB_sc(
        [hs_d2p[0], hs_d2p[1]], sb_d, db_d, cnt_d, ex0_d, ex1_d)

    hs2, es2 = _stage5d(raw_d1, den_d1, g1p_b, lin1_W, lin1_b, g2d_W, as2)
    ed2 = _stage5p(raw_p1, den_p1, g1d_b, lin1_W, lin1_b, vd2)

    raw2, den2 = _gat_pass_sc([hs2], es2.T, ed2.T, sb_d, db_d, cnt_d)

    return _stage8(raw2[0], den2.reshape(NPAD, 1),
                   g2d_b, lin2_W, lin2_b, lin3_W, lin3_b)


# speedup vs baseline: 1.0155x; 1.0155x over previous
import functools
import jax
import jax.numpy as jnp
from jax import lax
from jax.experimental import pallas as pl
from jax.experimental.pallas import tpu as pltpu
from jax.experimental.pallas import tpu_sc as plsc

N = 10000
E = 160000
H = 4
C = 128
DHC = H * C
CH = 128
OUT = 64
DIN = 256

BM = 256

NC = 2
NS = 16
NW = NC * NS
ROWS = 314
RSUB = ROWS // 2
NPAD = NW * ROWS
FCH = 8000
NCHUNK = E // FCH
FCH2 = 2048
CAP = E + 4096
ECH = 128
ECHA = 512
ECHB = 128
DENW = 320


def _grid(n):
    return (n + BM - 1) // BM


def _s1_body(xp_ref, xd_ref, Wsp_ref, Wsd_ref, Asp_ref, Asd_ref,
             vdp_ref, vdd_ref,
             hsp_ref, esp_ref, edp_ref, hsd_ref, esd_ref, edd_ref):
    xp = xp_ref[...]
    xd = xd_ref[...]
    hs_p = jnp.dot(xp, Wsp_ref[...], preferred_element_type=jnp.float32)
    hs_d = jnp.dot(xd, Wsd_ref[...], preferred_element_type=jnp.float32)
    for p in range(2):
        hsp_ref[p] = hs_p[:, p * 2 * C:(p + 1) * 2 * C]
        hsd_ref[p] = hs_d[:, p * 2 * C:(p + 1) * 2 * C]
    esp_ref[...] = jnp.dot(hs_p, Asp_ref[...], preferred_element_type=jnp.float32)
    esd_ref[...] = jnp.dot(hs_d, Asd_ref[...], preferred_element_type=jnp.float32)
    edp_ref[...] = jnp.dot(xd, vdp_ref[...], preferred_element_type=jnp.float32)
    edd_ref[...] = jnp.dot(xp, vdd_ref[...], preferred_element_type=jnp.float32)


def _stage1(xp, xd, Wsp, Wsd, Asp, Asd, vdp, vdd):
    g = _grid(N)
    full = lambda shape: pl.BlockSpec(shape, lambda i: (0,) * len(shape))
    row2 = pl.BlockSpec((BM, DIN), lambda i: (i, 0))
    outs = (
        jax.ShapeDtypeStruct((2, N, 2 * C), jnp.float32),
        jax.ShapeDtypeStruct((N, H), jnp.float32),
        jax.ShapeDtypeStruct((N, H), jnp.float32),
        jax.ShapeDtypeStruct((2, N, 2 * C), jnp.float32),
        jax.ShapeDtypeStruct((N, H), jnp.float32),
        jax.ShapeDtypeStruct((N, H), jnp.float32),
    )
    hs_spec = pl.BlockSpec((2, BM, 2 * C), lambda i: (0, i, 0))
    sc_spec = pl.BlockSpec((BM, H), lambda i: (i, 0))
    return pl.pallas_call(
        _s1_body,
        grid=(g,),
        in_specs=[row2, row2, full((DIN, DHC)), full((DIN, DHC)),
                  full((DHC, H)), full((DHC, H)), full((DIN, H)), full((DIN, H))],
        out_specs=(hs_spec, sc_spec, sc_spec, hs_spec, sc_spec, sc_spec),
        out_shape=outs,
    )(xp, xd, Wsp, Wsd, Asp, Asd, vdp, vdd)


def _s5d_body(raw_ref, den_ref, b_ref, W_ref, lb_ref, W2_ref, as2_ref,
              hs2_ref, es2_ref):
    parts = []
    for h in range(H):
        den = den_ref[h][:, None] + 1e-16
        pr, sub = divmod(h, 2)
        parts.append(raw_ref[pr][:, sub * C:(sub + 1) * C] / den
                     + b_ref[pl.ds(h * C, C)][None, :])
    d1 = jnp.concatenate(parts, axis=1)
    y = jax.nn.relu(d1 + jnp.dot(d1, W_ref[...], preferred_element_type=jnp.float32)
                    + lb_ref[...][None, :])
    hs2 = jnp.dot(y, W2_ref[...], preferred_element_type=jnp.float32)
    hs2_ref[...] = hs2
    es2_ref[...] = jnp.dot(hs2, as2_ref[...], preferred_element_type=jnp.float32)


def _stage5d(raw, den, gb, W, lb, W2, as2):
    g = _grid(N)
    npad = raw.shape[1]
    full = lambda shape: pl.BlockSpec(shape, lambda i: (0,) * len(shape))
    return pl.pallas_call(
        _s5d_body,
        grid=(g,),
        in_specs=[pl.BlockSpec((2, BM, 2 * C), lambda i: (0, i, 0)),
                  pl.BlockSpec((H, BM), lambda i: (0, i)),
                  full((DHC,)), full((DHC, DHC)), full((DHC,)),
                  full((DHC, CH)), full((CH, 1))],
        out_specs=(pl.BlockSpec((BM, CH), lambda i: (i, 0)),
                   pl.BlockSpec((BM, 1), lambda i: (i, 0))),
        out_shape=(jax.ShapeDtypeStruct((N, CH), jnp.float32),
                   jax.ShapeDtypeStruct((N, 1), jnp.float32)),
    )(raw, den, gb, W, lb, W2, as2)


def _s5p_body(raw_ref, den_ref, b_ref, W_ref, lb_ref, v2_ref, ed2_ref):
    parts = []
    for h in range(H):
        den = den_ref[h][:, None] + 1e-16
        pr, sub = divmod(h, 2)
        parts.append(raw_ref[pr][:, sub * C:(sub + 1) * C] / den
                     + b_ref[pl.ds(h * C, C)][None, :])
    p1 = jnp.concatenate(parts, axis=1)
    y = jax.nn.relu(p1 + jnp.dot(p1, W_ref[...], preferred_element_type=jnp.float32)
                    + lb_ref[...][None, :])
    ed2_ref[...] = jnp.dot(y, v2_ref[...], preferred_element_type=jnp.float32)


def _stage5p(raw, den, gb, W, lb, v2):
    g = _grid(N)
    full = lambda shape: pl.BlockSpec(shape, lambda i: (0,) * len(shape))
    return pl.pallas_call(
        _s5p_body,
        grid=(g,),
        in_specs=[pl.BlockSpec((2, BM, 2 * C), lambda i: (0, i, 0)),
                  pl.BlockSpec((H, BM), lambda i: (0, i)),
                  full((DHC,)), full((DHC, DHC)), full((DHC,)), full((DHC, 1))],
        out_specs=pl.BlockSpec((BM, 1), lambda i: (i, 0)),
        out_shape=jax.ShapeDtypeStruct((N, 1), jnp.float32),
    )(raw, den, gb, W, lb, v2)


def _s8_body(raw_ref, den_ref, b2_ref, W2_ref, lb2_ref, W3_ref, lb3_ref, out_ref):
    p2 = raw_ref[...] / (den_ref[...] + 1e-16) + b2_ref[...][None, :]
    p2 = jax.nn.relu(p2 + jnp.dot(p2, W2_ref[...], preferred_element_type=jnp.float32)
                     + lb2_ref[...][None, :])
    lg = jnp.dot(p2, W3_ref[...], preferred_element_type=jnp.float32) + lb3_ref[...][None, :]
    m = jnp.max(lg, axis=1, keepdims=True)
    ex = jnp.exp(lg - m)
    out_ref[...] = lg - m - jnp.log(jnp.sum(ex, axis=1, keepdims=True))


def _stage8(raw2, den2, b2, W2, lb2, W3, lb3):
    g = _grid(N)
    full = lambda shape: pl.BlockSpec(shape, lambda i: (0,) * len(shape))
    return pl.pallas_call(
        _s8_body,
        grid=(g,),
        in_specs=[pl.BlockSpec((BM, CH), lambda i: (i, 0)),
                  pl.BlockSpec((BM, 1), lambda i: (i, 0)),
                  full((CH,)), full((CH, CH)), full((CH,)),
                  full((CH, OUT)), full((OUT,))],
        out_specs=pl.BlockSpec((BM, OUT), lambda i: (i, 0)),
        out_shape=jax.ShapeDtypeStruct((N, OUT), jnp.float32),
    )(raw2, den2, b2, W2, lb2, W3, lb3)


def _filter_sc(s, d):
    mesh = plsc.VectorSubcoreMesh(core_axis_name="c", subcore_axis_name="s")

    @functools.partial(
        pl.kernel,
        out_type=(jax.ShapeDtypeStruct((NW * CAP,), jnp.int32),
                  jax.ShapeDtypeStruct((NW * CAP,), jnp.int32),
                  jax.ShapeDtypeStruct((NW * 16,), jnp.int32),
                  jax.ShapeDtypeStruct((NW * 2 * CAP,), jnp.int32),
                  jax.ShapeDtypeStruct((NW * 2 * CAP,), jnp.int32),
                  jax.ShapeDtypeStruct((NW * 16,), jnp.int32)),
        mesh=mesh,
        compiler_params=pltpu.CompilerParams(needs_layout_passes=False),
        scratch_types=[pltpu.VMEM((FCH,), jnp.int32),
                       pltpu.VMEM((FCH,), jnp.int32),
                       pltpu.VMEM((FCH + 32,), jnp.int32),
                       pltpu.VMEM((FCH + 32,), jnp.int32)],
    )
    def filt(s_hbm, d_hbm, sb_hbm, db_hbm, cnt_hbm, sb2_hbm, db2_hbm,
             cnt2_hbm, s_in, d_in, st_s, st_d):
        wid = lax.axis_index("s") * NC + lax.axis_index("c")
        lo = wid * ROWS
        wbase = wid * CAP
        w2base = wid * 2 * CAP
        lane = lax.iota(jnp.int32, 16)

        def chunk_body(ck, cursor):
            pltpu.sync_copy(s_hbm.at[pl.ds(ck * FCH, FCH)], s_in)
            pltpu.sync_copy(d_hbm.at[pl.ds(ck * FCH, FCH)], d_in)

            def vec_body(i, cnt):
                sv = s_in[pl.ds(i * 16, 16)]
                dv = d_in[pl.ds(i * 16, 16)]
                m = (dv >= lo) & (dv < lo + ROWS)
                mi = m.astype(jnp.int32)
                excl = plsc.cumsum(mi) - mi
                idx = jnp.where(m, cnt + excl, FCH + 16)
                plsc.store_scatter(st_s, [idx], sv)
                plsc.store_scatter(st_d, [idx], dv - lo)
                return cnt + jnp.sum(mi)

            cnt = lax.fori_loop(0, FCH // 16, vec_body, jnp.int32(0))
            st_s[pl.ds(cnt, 16)] = jnp.zeros((16,), jnp.int32)
            st_d[pl.ds(cnt, 16)] = jnp.full((16,), ROWS, jnp.int32)
            cnt_pad = ((cnt + 15) // 16) * 16
            off = pl.multiple_of(wbase + cursor, 16)
            pltpu.sync_copy(st_s.at[pl.ds(0, FCH + 16)],
                            sb_hbm.at[pl.ds(off, FCH + 16)])
            pltpu.sync_copy(st_d.at[pl.ds(0, FCH + 16)],
                            db_hbm.at[pl.ds(off, FCH + 16)])
            return cursor + cnt_pad

        cursor = lax.fori_loop(0, NCHUNK, chunk_body, jnp.int32(0))
        for j in range(ECH // 16):
            st_s[pl.ds(j * 16, 16)] = jnp.zeros((16,), jnp.int32)
            st_d[pl.ds(j * 16, 16)] = jnp.full((16,), ROWS, jnp.int32)
        off = pl.multiple_of(wbase + cursor, 16)
        pltpu.sync_copy(st_s.at[pl.ds(0, ECH)], sb_hbm.at[pl.ds(off, ECH)])
        pltpu.sync_copy(st_d.at[pl.ds(0, ECH)], db_hbm.at[pl.ds(off, ECH)])
        nch = (cursor + ECH - 1) // ECH
        st_s[pl.ds(0, 16)] = jnp.full((16,), 1, jnp.int32) * nch
        pltpu.sync_copy(st_s.at[pl.ds(0, 16)],
                        cnt_hbm.at[pl.ds(pl.multiple_of(wid * 16, 16), 16)])

        nche = nch * ECH

        def rb_chunk(ck, curs):
            c0, c1 = curs
            roff = pl.multiple_of(wbase + ck * FCH2, 16)
            pltpu.sync_copy(sb_hbm.at[pl.ds(roff, FCH2)],
                            s_in.at[pl.ds(0, FCH2)])
            pltpu.sync_copy(db_hbm.at[pl.ds(roff, FCH2)],
                            d_in.at[pl.ds(0, FCH2)])

            def rb_vec(j, carry):
                l0, l1 = carry
                sv = s_in[pl.ds(j * 16, 16)]
                dl = d_in[pl.ds(j * 16, 16)]
                valid = (ck * FCH2 + j * 16 + lane) < nche
                m0 = valid & (dl < RSUB)
                m1 = valid & (dl >= RSUB) & (dl < ROWS)
                mi0 = m0.astype(jnp.int32)
                mi1 = m1.astype(jnp.int32)
                e0 = plsc.cumsum(mi0) - mi0
                e1 = plsc.cumsum(mi1) - mi1
                i0 = jnp.where(m0, l0 + e0, FCH2)
                i1 = jnp.where(m1, 4096 + l1 + e1, 4096 + FCH2)
                plsc.store_scatter(st_s, [i0], sv)
                plsc.store_scatter(st_d, [i0], dl)
                plsc.store_scatter(st_s, [i1], sv)
                plsc.store_scatter(st_d, [i1], dl - RSUB)
                return l0 + jnp.sum(mi0), l1 + jnp.sum(mi1)

            l0, l1 = lax.fori_loop(0, FCH2 // 16, rb_vec,
                                   (jnp.int32(0), jnp.int32(0)))
            st_s[pl.ds(l0, 16)] = jnp.zeros((16,), jnp.int32)
            st_d[pl.ds(l0, 16)] = jnp.full((16,), RSUB, jnp.int32)
            st_s[pl.ds(4096 + l1, 16)] = jnp.zeros((16,), jnp.int32)
            st_d[pl.ds(4096 + l1, 16)] = jnp.full((16,), RSUB, jnp.int32)
            o0 = pl.multiple_of(w2base + c0, 16)
            o1 = pl.multiple_of(w2base + CAP + c1, 16)
            pltpu.sync_copy(st_s.at[pl.ds(0, FCH2 + 16)],
                            sb2_hbm.at[pl.ds(o0, FCH2 + 16)])
            pltpu.sync_copy(st_d.at[pl.ds(0, FCH2 + 16)],
                            db2_hbm.at[pl.ds(o0, FCH2 + 16)])
            pltpu.sync_copy(st_s.at[pl.ds(4096, FCH2 + 16)],
                            sb2_hbm.at[pl.ds(o1, FCH2 + 16)])
            pltpu.sync_copy(st_d.at[pl.ds(4096, FCH2 + 16)],
                            db2_hbm.at[pl.ds(o1, FCH2 + 16)])
            return (c0 + ((l0 + 15) // 16) * 16, c1 + ((l1 + 15) // 16) * 16)

        nck2 = (nche + FCH2 - 1) // FCH2
        c0, c1 = lax.fori_loop(0, nck2, rb_chunk,
                               (jnp.int32(0), jnp.int32(0)))
        for j in range(ECHB // 16):
            st_s[pl.ds(j * 16, 16)] = jnp.zeros((16,), jnp.int32)
            st_d[pl.ds(j * 16, 16)] = jnp.full((16,), RSUB, jnp.int32)
        pltpu.sync_copy(st_s.at[pl.ds(0, ECHB)],
                        sb2_hbm.at[pl.ds(pl.multiple_of(w2base + c0, 16), ECHB)])
        pltpu.sync_copy(st_d.at[pl.ds(0, ECHB)],
                        db2_hbm.at[pl.ds(pl.multiple_of(w2base + c0, 16), ECHB)])
        pltpu.sync_copy(st_s.at[pl.ds(0, ECHB)],
                        sb2_hbm.at[pl.ds(pl.multiple_of(w2base + CAP + c1, 16), ECHB)])
        pltpu.sync_copy(st_d.at[pl.ds(0, ECHB)],
                        db2_hbm.at[pl.ds(pl.multiple_of(w2base + CAP + c1, 16), ECHB)])
        n0 = (c0 + ECHB - 1) // ECHB
        n1 = (c1 + ECHB - 1) // ECHB
        st_s[pl.ds(0, 16)] = jnp.where(lane < 1, n0, n1)
        pltpu.sync_copy(st_s.at[pl.ds(0, 16)],
                        cnt2_hbm.at[pl.ds(pl.multiple_of(wid * 16, 16), 16)])

    return filt(s, d)


def _gat_pass_sc(hs_list, es_T, ed_T, sb, db, cnt):
    nh = len(hs_list)
    mesh = plsc.VectorSubcoreMesh(core_axis_name="c", subcore_axis_name="s")

    @functools.partial(
        pl.kernel,
        out_type=(jax.ShapeDtypeStruct((nh * NPAD * C,), jnp.float32),
                  jax.ShapeDtypeStruct((nh * NW * DENW,), jnp.float32)),
        mesh=mesh,
        compiler_params=pltpu.CompilerParams(needs_layout_passes=False),
        scratch_types=[pltpu.VMEM(((ROWS + 1) * C,), jnp.float32),
                       pltpu.VMEM((ECH, C), jnp.float32),
                       pltpu.VMEM((ECH, C), jnp.float32),
                       pltpu.VMEM((N,), jnp.float32),
                       pltpu.VMEM((N,), jnp.float32),
                       pltpu.VMEM((DENW,), jnp.float32),
                       pltpu.VMEM((ECH,), jnp.int32),
                       pltpu.VMEM((ECH,), jnp.int32),
                       pltpu.VMEM((ECH,), jnp.int32),
                       pltpu.VMEM((ECH,), jnp.int32),
                       pltpu.VMEM((ECH,), jnp.float32),
                       pltpu.VMEM((16,), jnp.int32),
                       pltpu.SemaphoreType.DMA,
                       pltpu.SemaphoreType.DMA],
    )
    def gat(*refs):
        hs_refs = refs[:nh]
        es_hbm, ed_hbm, sb_hbm, db_hbm, cnt_hbm, raw_hbm, den_hbm = refs[nh:nh + 7]
        (out_f, rows0, rows1, es_v, ed_v, den_v, s_v0, s_v1, d_v0, d_v1,
         ex_v, cnt_v, sem0, sem1) = refs[nh + 7:]
        wid = lax.axis_index("s") * NC + lax.axis_index("c")
        lo = wid * ROWS
        wbase = wid * CAP
        pltpu.sync_copy(cnt_hbm.at[pl.ds(pl.multiple_of(wid * 16, 16), 16)],
                        cnt_v)
        nch = cnt_v[pl.ds(0, 16)][0]
        for h in range(nh):
            hs_ref = hs_refs[h]
            pltpu.sync_copy(es_hbm.at[pl.ds(h * N, N)], es_v)
            pltpu.sync_copy(ed_hbm.at[pl.ds(h * N, N)], ed_v)

            def zbody(i, _):
                out_f[pl.ds(i * 16, 16)] = jnp.zeros((16,), jnp.float32)
                return 0

            lax.fori_loop(0, (ROWS + 1) * C // 16, zbody, 0)
            for i in range(DENW // 16):
                den_v[pl.ds(i * 16, 16)] = jnp.zeros((16,), jnp.float32)

            def fetch(ck, s_v, d_v, rows, sem):
                @pl.when(ck < nch)
                def _():
                    eoff = pl.multiple_of(wbase + ck * ECH, 16)
                    pltpu.sync_copy(sb_hbm.at[pl.ds(eoff, ECH)], s_v)
                    pltpu.sync_copy(db_hbm.at[pl.ds(eoff, ECH)], d_v)
                    pltpu.make_async_copy(hs_ref.at[s_v], rows, sem).start()

            def process(ck, s_v, d_v, rows, sem):
                @pl.when(ck < nch)
                def _():
                    pltpu.make_async_copy(hs_ref.at[s_v], rows, sem).wait()

                    def vec_body(j, _):
                        sv = s_v[pl.ds(j * 16, 16)]
                        dv = d_v[pl.ds(j * 16, 16)]
                        esg = plsc.load_gather(es_v, [sv])
                        edi = jnp.minimum(dv + lo, N - 1)
                        edg = plsc.load_gather(ed_v, [edi])
                        e = esg + edg
                        e = jnp.where(e >= 0.0, e, 0.2 * e)
                        exv = jnp.exp(e)
                        plsc.addupdate_scatter(den_v, [dv], exv)
                        ex_v[pl.ds(j * 16, 16)] = exv
                        return 0

                    lax.fori_loop(0, ECH // 16, vec_body, 0)

                    def fma_body(k, _):
                        dv16 = d_v[pl.ds(k * 16, 16)]
                        ex16 = ex_v[pl.ds(k * 16, 16)]
                        for l in range(16):
                            base = dv16[l] * C
                            exb = jnp.full((16,), ex16[l])
                            for j in range(C // 16):
                                plsc.addupdate(
                                    out_f.at[pl.ds(base + j * 16, 16)],
                                    exb * rows[k * 16 + l, pl.ds(j * 16, 16)])
                        return 0

                    lax.fori_loop(0, ECH // 16, fma_body, 0)

            fetch(jnp.int32(0), s_v0, d_v0, rows0, sem0)

            def pair_body(k, _):
                c0 = k * 2
                fetch(c0 + 1, s_v1, d_v1, rows1, sem1)
                process(c0, s_v0, d_v0, rows0, sem0)
                fetch(c0 + 2, s_v0, d_v0, rows0, sem0)
                process(c0 + 1, s_v1, d_v1, rows1, sem1)
                return 0

            lax.fori_loop(0, (nch + 1) // 2, pair_body, 0)
            roff = pl.multiple_of(h * NPAD * C + lo * C, 16)
            pltpu.sync_copy(out_f.at[pl.ds(0, ROWS * C)],
                            raw_hbm.at[pl.ds(roff, ROWS * C)])
            doff = pl.multiple_of(h * NW * DENW + wid * DENW, 16)
            pltpu.sync_copy(den_v, den_hbm.at[pl.ds(doff, DENW)])

    raw_f, den_f = gat(*hs_list, es_T.reshape(-1), ed_T.reshape(-1), sb, db, cnt)
    raw = raw_f.reshape(nh, NPAD, C)
    den = den_f.reshape(nh, NW, DENW)[:, :, :ROWS].reshape(nh, NPAD)
    return raw, den


def _gat_passA_sc(es_i, ed_i, sb, db, cnt):
    mesh = plsc.VectorSubcoreMesh(core_axis_name="c", subcore_axis_name="s")

    @functools.partial(
        pl.kernel,
        out_type=(jax.ShapeDtypeStruct((NW * 2 * CAP * 2,), jnp.float32),
                  jax.ShapeDtypeStruct((NW * 2 * CAP * 2,), jnp.float32),
                  jax.ShapeDtypeStruct((NW * 4 * DENW,), jnp.float32)),
        mesh=mesh,
        compiler_params=pltpu.CompilerParams(needs_layout_passes=False),
        scratch_types=[pltpu.VMEM((4 * N,), jnp.float32),
                       pltpu.VMEM((4 * N,), jnp.float32),
                       pltpu.VMEM((ECHA,), jnp.int32),
                       pltpu.VMEM((ECHA,), jnp.int32),
                       pltpu.VMEM((2 * ECHA,), jnp.float32),
                       pltpu.VMEM((2 * ECHA,), jnp.float32),
                       pltpu.VMEM((4 * DENW,), jnp.float32),
                       pltpu.VMEM((16,), jnp.int32)],
    )
    def pA(es_hbm, ed_hbm, sb_hbm, db_hbm, cnt_hbm, ex0_hbm, ex1_hbm, den_hbm,
           es_v, ed_v, s_v, d_v, x0_v, x1_v, den_v, cnt_v):
        wid = lax.axis_index("s") * NC + lax.axis_index("c")
        lo = wid * ROWS
        pltpu.sync_copy(cnt_hbm.at[pl.ds(pl.multiple_of(wid * 16, 16), 16)],
                        cnt_v)
        pltpu.sync_copy(es_hbm, es_v)
        pltpu.sync_copy(ed_hbm, ed_v)
        for i in range(4 * DENW // 16):
            den_v[pl.ds(i * 16, 16)] = jnp.zeros((16,), jnp.float32)
        lane = lax.iota(jnp.int32, 16)
        cntv16 = cnt_v[pl.ds(0, 16)]

        def sub_body(b, _):
            base2 = wid * 2 * CAP + b * CAP
            nche = jnp.where(b == 0, cntv16[0], cntv16[1]) * ECHB
            ncha = (nche + ECHA - 1) // ECHA

            def chunk_body(ck, _):
                eoff = pl.multiple_of(base2 + ck * ECHA, 16)
                pltpu.sync_copy(sb_hbm.at[pl.ds(eoff, ECHA)], s_v)
                pltpu.sync_copy(db_hbm.at[pl.ds(eoff, ECHA)], d_v)

                def vec_body(j, _):
                    valid = (ck * ECHA + j * 16 + lane) < nche
                    sv = jnp.where(valid, s_v[pl.ds(j * 16, 16)], 0)
                    dsub = jnp.where(valid, d_v[pl.ds(j * 16, 16)], RSUB)
                    dv = jnp.where(dsub >= RSUB, ROWS, b * RSUB + dsub)
                    edi = jnp.minimum(dv + lo, N - 1) * 4
                    svi = sv * 4
                    for h in range(4):
                        esg = plsc.load_gather(es_v, [svi + h])
                        edg = plsc.load_gather(ed_v, [edi + h])
                        e = esg + edg
                        e = jnp.where(e >= 0.0, e, 0.2 * e)
                        exv = jnp.exp(e)
                        plsc.addupdate_scatter(den_v, [dv + h * DENW], exv)
                        xst = x0_v if h < 2 else x1_v
                        plsc.store_scatter(xst,
                                           [lane * 2 + (j * 32 + (h % 2))],
                                           exv)
                    return 0

                lax.fori_loop(0, ECHA // 16, vec_body, 0)
                xoff = pl.multiple_of(2 * (base2 + ck * ECHA), 16)
                pltpu.sync_copy(x0_v, ex0_hbm.at[pl.ds(xoff, 2 * ECHA)])
                pltpu.sync_copy(x1_v, ex1_hbm.at[pl.ds(xoff, 2 * ECHA)])
                return 0

            lax.fori_loop(0, ncha, chunk_body, 0)
            return 0

        lax.fori_loop(0, 2, sub_body, 0)
        doff = pl.multiple_of(wid * 4 * DENW, 16)
        pltpu.sync_copy(den_v, den_hbm.at[pl.ds(doff, 4 * DENW)])

    ex0, ex1, den_f = pA(es_i, ed_i, sb, db, cnt)
    den = (den_f.reshape(NW, 4, DENW)[:, :, :ROWS]
           .transpose(1, 0, 2).reshape(4, NPAD))
    return ex0, ex1, den


def _gat_passB_sc(hs_pairs, sb, db, cnt, ex0, ex1):
    mesh = plsc.VectorSubcoreMesh(core_axis_name="c", subcore_axis_name="s")
    C2 = 2 * C

    @functools.partial(
        pl.kernel,
        out_type=jax.ShapeDtypeStruct((2 * NPAD * C2,), jnp.float32),
        mesh=mesh,
        compiler_params=pltpu.CompilerParams(needs_layout_passes=False),
        scratch_types=[pltpu.VMEM(((RSUB + 1) * C2,), jnp.float32),
                       pltpu.VMEM((ECHB, C2), jnp.float32),
                       pltpu.VMEM((ECHB, C2), jnp.float32),
                       pltpu.VMEM((ECHB,), jnp.int32),
                       pltpu.VMEM((ECHB,), jnp.int32),
                       pltpu.VMEM((ECHB,), jnp.int32),
                       pltpu.VMEM((ECHB,), jnp.int32),
                       pltpu.VMEM((2 * ECHB,), jnp.float32),
                       pltpu.VMEM((2 * ECHB,), jnp.float32),
                       pltpu.VMEM((16,), jnp.int32),
                       pltpu.SemaphoreType.DMA,
                       pltpu.SemaphoreType.DMA],
    )
    def pB(hs0_hbm, hs1_hbm, sb_hbm, db_hbm, cnt_hbm, ex0_hbm, ex1_hbm,
           raw_hbm, out_f, rows0, rows1, s_v0, s_v1, d_v0, d_v1, x_v0, x_v1,
           cnt_v, sem0, sem1):
        wid = lax.axis_index("s") * NC + lax.axis_index("c")
        lo = wid * ROWS
        pltpu.sync_copy(cnt_hbm.at[pl.ds(pl.multiple_of(wid * 16, 16), 16)],
                        cnt_v)
        cntv16 = cnt_v[pl.ds(0, 16)]
        for p in range(2):
            hs_ref = (hs0_hbm, hs1_hbm)[p]
            ex_ref = (ex0_hbm, ex1_hbm)[p]

            def sub_body(b, _):
                base2 = wid * 2 * CAP + b * CAP
                nchb = jnp.where(b == 0, cntv16[0], cntv16[1])

                def zbody(i, _):
                    out_f[pl.ds(i * 16, 16)] = jnp.zeros((16,), jnp.float32)
                    return 0

                lax.fori_loop(0, (RSUB + 1) * C2 // 16, zbody, 0)

                def fetch(ck, s_v, d_v, x_v, rows, sem):
                    @pl.when(ck < nchb)
                    def _():
                        eoff = pl.multiple_of(base2 + ck * ECHB, 16)
                        pltpu.sync_copy(sb_hbm.at[pl.ds(eoff, ECHB)], s_v)
                        pltpu.sync_copy(db_hbm.at[pl.ds(eoff, ECHB)], d_v)
                        xoff = pl.multiple_of(2 * (base2 + ck * ECHB), 16)
                        pltpu.sync_copy(ex_ref.at[pl.ds(xoff, 2 * ECHB)], x_v)
                        pltpu.make_async_copy(hs_ref.at[s_v], rows,
                                              sem).start()

                def process(ck, s_v, d_v, x_v, rows, sem):
                    @pl.when(ck < nchb)
                    def _():
                        pltpu.make_async_copy(hs_ref.at[s_v], rows,
                                              sem).wait()

                        def fma_body(k, _):
                            dv16 = d_v[pl.ds(k * 16, 16)]
                            xa = x_v[pl.ds(k * 32, 16)]
                            xb = x_v[pl.ds(k * 32 + 16, 16)]
                            for l in range(16):
                                e = k * 16 + l
                                xv = xa if l < 8 else xb
                                base = dv16[l] * C2
                                for hh in range(2):
                                    exb = jnp.full((16,),
                                                   xv[(l % 8) * 2 + hh])
                                    for j in range(C // 16):
                                        off = hh * C + j * 16
                                        plsc.addupdate(
                                            out_f.at[pl.ds(base + off, 16)],
                                            exb * rows[e, pl.ds(off, 16)])
                            return 0

                        lax.fori_loop(0, ECHB // 16, fma_body, 0)

                fetch(jnp.int32(0), s_v0, d_v0, x_v0, rows0, sem0)

                def pair_body(k, _):
                    c0 = k * 2
                    fetch(c0 + 1, s_v1, d_v1, x_v1, rows1, sem1)
                    process(c0, s_v0, d_v0, x_v0, rows0, sem0)
                    fetch(c0 + 2, s_v0, d_v0, x_v0, rows0, sem0)
                    process(c0 + 1, s_v1, d_v1, x_v1, rows1, sem1)
                    return 0

                lax.fori_loop(0, (nchb + 1) // 2, pair_body, 0)
                roff = pl.multiple_of(
                    p * NPAD * C2 + (lo + b * RSUB) * C2, 16)
                pltpu.sync_copy(out_f.at[pl.ds(0, RSUB * C2)],
                                raw_hbm.at[pl.ds(roff, RSUB * C2)])
                return 0

            lax.fori_loop(0, 2, sub_body, 0)

    raw_f = pB(hs_pairs[0], hs_pairs[1], sb, db, cnt, ex0, ex1)
    return raw_f.reshape(2, NPAD, C2)


def _blockdiag_a(a):
    hh = a.shape[0]
    eye = jnp.eye(hh, dtype=a.dtype)
    A = eye[:, :, None] * a[:, None, :]
    return A.transpose(1, 2, 0).reshape(hh * a.shape[1], hh)


def kernel(x_person, x_diagnosis, edge_index_p2d, edge_index_d2p, edge_attr_p2d,
           g1p_Ws, g1p_Wd, g1p_as, g1p_ad, g1p_b,
           g1d_Ws, g1d_Wd, g1d_as, g1d_ad, g1d_b,
           g2p_W, g2p_as, g2p_ad, g2p_b,
           g2d_W, g2d_as, g2d_ad, g2d_b,
           lin1_W, lin1_b, lin2_W, lin2_b, lin3_W, lin3_b,
           em1_W, em1_b, em2_W, em2_b):
    Asp = _blockdiag_a(g1p_as)
    Adp = _blockdiag_a(g1p_ad)
    Asd = _blockdiag_a(g1d_as)
    Add = _blockdiag_a(g1d_ad)
    vdp = g1p_Wd @ Adp
    vdd = g1d_Wd @ Add
    as2 = g2d_as.reshape(CH, 1)
    vd2 = g2d_W @ g2d_ad.reshape(CH, 1)

    hs_p2d, es_p2d, ed_p2d, hs_d2p, es_d2p, ed_d2p = _stage1(
        x_person, x_diagnosis, g1p_Ws, g1d_Ws, Asp, Asd, vdp, vdd)

    s_p2d = edge_index_p2d[0]
    d_p2d = edge_index_p2d[1]
    s_d2p = edge_index_d2p[0]
    d_d2p = edge_index_d2p[1]

    sb_p, db_p, cnt_p, sb2_p, db2_p, cnt2_p = _filter_sc(s_p2d, d_p2d)
    sb_d, db_d, cnt_d, sb2_d, db2_d, cnt2_d = _filter_sc(s_d2p, d_d2p)

    ex0_p, ex1_p, den_d1 = _gat_passA_sc(
        es_p2d.reshape(-1), ed_p2d.reshape(-1), sb2_p, db2_p, cnt2_p)
    raw_d1 = _gat_passB_sc(
        [hs_p2d[0], hs_p2d[1]], sb2_p, db2_p, cnt2_p, ex0_p, ex1_p)
    ex0_d, ex1_d, den_p1 = _gat_passA_sc(
        es_d2p.reshape(-1), ed_d2p.reshape(-1), sb2_d, db2_d, cnt2_d)
    raw_p1 = _gat_passB_sc(
        [hs_d2p[0], hs_d2p[1]], sb2_d, db2_d, cnt2_d, ex0_d, ex1_d)

    hs2, es2 = _stage5d(raw_d1, den_d1, g1p_b, lin1_W, lin1_b, g2d_W, as2)
    ed2 = _stage5p(raw_p1, den_p1, g1d_b, lin1_W, lin1_b, vd2)

    raw2, den2 = _gat_pass_sc([hs2], es2.T, ed2.T, sb_d, db_d, cnt_d)

    return _stage8(raw2[0], den2.reshape(NPAD, 1),
                   g2d_b, lin2_W, lin2_b, lin3_W, lin3_b)


# speedup vs baseline: 1.0467x; 1.0308x over previous
import functools
import jax
import jax.numpy as jnp
from jax import lax
from jax.experimental import pallas as pl
from jax.experimental.pallas import tpu as pltpu
from jax.experimental.pallas import tpu_sc as plsc

N = 10000
E = 160000
H = 4
C = 128
DHC = H * C
CH = 128
OUT = 64
DIN = 256

BM = 256

NC = 2
NS = 16
NW = NC * NS
ROWS = 314
RSUB = ROWS // 2
NPAD = NW * ROWS
FCH = 8000
NCHUNK = E // FCH
FCH2 = 2048
CAP = E + 4096
ECH = 128
ECHA = 512
ECHB = 128
SCH = 8
DENW = 320


def _grid(n):
    return (n + BM - 1) // BM


def _s1_body(xp_ref, xd_ref, Wsp_ref, Wsd_ref, Asp_ref, Asd_ref,
             vdp_ref, vdd_ref,
             hsp_ref, esp_ref, edp_ref, hsd_ref, esd_ref, edd_ref):
    xp = xp_ref[...]
    xd = xd_ref[...]
    hs_p = jnp.dot(xp, Wsp_ref[...], preferred_element_type=jnp.float32)
    hs_d = jnp.dot(xd, Wsd_ref[...], preferred_element_type=jnp.float32)
    for p in range(2):
        hsp_ref[p] = hs_p[:, p * 2 * C:(p + 1) * 2 * C]
        hsd_ref[p] = hs_d[:, p * 2 * C:(p + 1) * 2 * C]
    esp_ref[...] = jnp.dot(hs_p, Asp_ref[...], preferred_element_type=jnp.float32)
    esd_ref[...] = jnp.dot(hs_d, Asd_ref[...], preferred_element_type=jnp.float32)
    edp_ref[...] = jnp.dot(xd, vdp_ref[...], preferred_element_type=jnp.float32)
    edd_ref[...] = jnp.dot(xp, vdd_ref[...], preferred_element_type=jnp.float32)


def _stage1(xp, xd, Wsp, Wsd, Asp, Asd, vdp, vdd):
    g = _grid(N)
    full = lambda shape: pl.BlockSpec(shape, lambda i: (0,) * len(shape))
    row2 = pl.BlockSpec((BM, DIN), lambda i: (i, 0))
    outs = (
        jax.ShapeDtypeStruct((2, N, 2 * C), jnp.float32),
        jax.ShapeDtypeStruct((N, H), jnp.float32),
        jax.ShapeDtypeStruct((N, H), jnp.float32),
        jax.ShapeDtypeStruct((2, N, 2 * C), jnp.float32),
        jax.ShapeDtypeStruct((N, H), jnp.float32),
        jax.ShapeDtypeStruct((N, H), jnp.float32),
    )
    hs_spec = pl.BlockSpec((2, BM, 2 * C), lambda i: (0, i, 0))
    sc_spec = pl.BlockSpec((BM, H), lambda i: (i, 0))
    return pl.pallas_call(
        _s1_body,
        grid=(g,),
        in_specs=[row2, row2, full((DIN, DHC)), full((DIN, DHC)),
                  full((DHC, H)), full((DHC, H)), full((DIN, H)), full((DIN, H))],
        out_specs=(hs_spec, sc_spec, sc_spec, hs_spec, sc_spec, sc_spec),
        out_shape=outs,
    )(xp, xd, Wsp, Wsd, Asp, Asd, vdp, vdd)


def _s5d_body(raw_ref, den_ref, b_ref, W_ref, lb_ref, W2_ref, as2_ref,
              hs2_ref, es2_ref):
    parts = []
    for h in range(H):
        den = den_ref[h][:, None] + 1e-16
        pr, sub = divmod(h, 2)
        parts.append(raw_ref[pr][:, sub * C:(sub + 1) * C] / den
                     + b_ref[pl.ds(h * C, C)][None, :])
    d1 = jnp.concatenate(parts, axis=1)
    y = jax.nn.relu(d1 + jnp.dot(d1, W_ref[...], preferred_element_type=jnp.float32)
                    + lb_ref[...][None, :])
    hs2 = jnp.dot(y, W2_ref[...], preferred_element_type=jnp.float32)
    hs2_ref[...] = hs2
    es2_ref[...] = jnp.dot(hs2, as2_ref[...], preferred_element_type=jnp.float32)


def _stage5d(raw, den, gb, W, lb, W2, as2):
    g = _grid(N)
    npad = raw.shape[1]
    full = lambda shape: pl.BlockSpec(shape, lambda i: (0,) * len(shape))
    return pl.pallas_call(
        _s5d_body,
        grid=(g,),
        in_specs=[pl.BlockSpec((2, BM, 2 * C), lambda i: (0, i, 0)),
                  pl.BlockSpec((H, BM), lambda i: (0, i)),
                  full((DHC,)), full((DHC, DHC)), full((DHC,)),
                  full((DHC, CH)), full((CH, 1))],
        out_specs=(pl.BlockSpec((BM, CH), lambda i: (i, 0)),
                   pl.BlockSpec((BM, 1), lambda i: (i, 0))),
        out_shape=(jax.ShapeDtypeStruct((N, CH), jnp.float32),
                   jax.ShapeDtypeStruct((N, 1), jnp.float32)),
    )(raw, den, gb, W, lb, W2, as2)


def _s5p_body(raw_ref, den_ref, b_ref, W_ref, lb_ref, v2_ref, ed2_ref):
    parts = []
    for h in range(H):
        den = den_ref[h][:, None] + 1e-16
        pr, sub = divmod(h, 2)
        parts.append(raw_ref[pr][:, sub * C:(sub + 1) * C] / den
                     + b_ref[pl.ds(h * C, C)][None, :])
    p1 = jnp.concatenate(parts, axis=1)
    y = jax.nn.relu(p1 + jnp.dot(p1, W_ref[...], preferred_element_type=jnp.float32)
                    + lb_ref[...][None, :])
    ed2_ref[...] = jnp.dot(y, v2_ref[...], preferred_element_type=jnp.float32)


def _stage5p(raw, den, gb, W, lb, v2):
    g = _grid(N)
    full = lambda shape: pl.BlockSpec(shape, lambda i: (0,) * len(shape))
    return pl.pallas_call(
        _s5p_body,
        grid=(g,),
        in_specs=[pl.BlockSpec((2, BM, 2 * C), lambda i: (0, i, 0)),
                  pl.BlockSpec((H, BM), lambda i: (0, i)),
                  full((DHC,)), full((DHC, DHC)), full((DHC,)), full((DHC, 1))],
        out_specs=pl.BlockSpec((BM, 1), lambda i: (i, 0)),
        out_shape=jax.ShapeDtypeStruct((N, 1), jnp.float32),
    )(raw, den, gb, W, lb, v2)


def _s8_body(raw_ref, den_ref, b2_ref, W2_ref, lb2_ref, W3_ref, lb3_ref, out_ref):
    p2 = raw_ref[...] / (den_ref[...] + 1e-16) + b2_ref[...][None, :]
    p2 = jax.nn.relu(p2 + jnp.dot(p2, W2_ref[...], preferred_element_type=jnp.float32)
                     + lb2_ref[...][None, :])
    lg = jnp.dot(p2, W3_ref[...], preferred_element_type=jnp.float32) + lb3_ref[...][None, :]
    m = jnp.max(lg, axis=1, keepdims=True)
    ex = jnp.exp(lg - m)
    out_ref[...] = lg - m - jnp.log(jnp.sum(ex, axis=1, keepdims=True))


def _stage8(raw2, den2, b2, W2, lb2, W3, lb3):
    g = _grid(N)
    full = lambda shape: pl.BlockSpec(shape, lambda i: (0,) * len(shape))
    return pl.pallas_call(
        _s8_body,
        grid=(g,),
        in_specs=[pl.BlockSpec((BM, CH), lambda i: (i, 0)),
                  pl.BlockSpec((BM, 1), lambda i: (i, 0)),
                  full((CH,)), full((CH, CH)), full((CH,)),
                  full((CH, OUT)), full((OUT,))],
        out_specs=pl.BlockSpec((BM, OUT), lambda i: (i, 0)),
        out_shape=jax.ShapeDtypeStruct((N, OUT), jnp.float32),
    )(raw2, den2, b2, W2, lb2, W3, lb3)


def _filter_sc(s, d):
    mesh = plsc.VectorSubcoreMesh(core_axis_name="c", subcore_axis_name="s")

    @functools.partial(
        pl.kernel,
        out_type=(jax.ShapeDtypeStruct((NW * CAP,), jnp.int32),
                  jax.ShapeDtypeStruct((NW * CAP,), jnp.int32),
                  jax.ShapeDtypeStruct((NW * 16,), jnp.int32),
                  jax.ShapeDtypeStruct((NW * 2 * CAP,), jnp.int32),
                  jax.ShapeDtypeStruct((NW * 2 * CAP,), jnp.int32),
                  jax.ShapeDtypeStruct((NW * 16,), jnp.int32)),
        mesh=mesh,
        compiler_params=pltpu.CompilerParams(needs_layout_passes=False),
        scratch_types=[pltpu.VMEM((FCH,), jnp.int32),
                       pltpu.VMEM((FCH,), jnp.int32),
                       pltpu.VMEM((FCH + 32,), jnp.int32),
                       pltpu.VMEM((FCH + 32,), jnp.int32)],
    )
    def filt(s_hbm, d_hbm, sb_hbm, db_hbm, cnt_hbm, sb2_hbm, db2_hbm,
             cnt2_hbm, s_in, d_in, st_s, st_d):
        wid = lax.axis_index("s") * NC + lax.axis_index("c")
        lo = wid * ROWS
        wbase = wid * CAP
        w2base = wid * 2 * CAP
        lane = lax.iota(jnp.int32, 16)

        def chunk_body(ck, cursor):
            pltpu.sync_copy(s_hbm.at[pl.ds(ck * FCH, FCH)], s_in)
            pltpu.sync_copy(d_hbm.at[pl.ds(ck * FCH, FCH)], d_in)

            def vec_body(i, cnt):
                sv = s_in[pl.ds(i * 16, 16)]
                dv = d_in[pl.ds(i * 16, 16)]
                m = (dv >= lo) & (dv < lo + ROWS)
                mi = m.astype(jnp.int32)
                excl = plsc.cumsum(mi) - mi
                idx = jnp.where(m, cnt + excl, FCH + 16)
                plsc.store_scatter(st_s, [idx], sv)
                plsc.store_scatter(st_d, [idx], dv - lo)
                return cnt + jnp.sum(mi)

            cnt = lax.fori_loop(0, FCH // 16, vec_body, jnp.int32(0))
            st_s[pl.ds(cnt, 16)] = jnp.zeros((16,), jnp.int32)
            st_d[pl.ds(cnt, 16)] = jnp.full((16,), ROWS, jnp.int32)
            cnt_pad = ((cnt + 15) // 16) * 16
            off = pl.multiple_of(wbase + cursor, 16)
            pltpu.sync_copy(st_s.at[pl.ds(0, FCH + 16)],
                            sb_hbm.at[pl.ds(off, FCH + 16)])
            pltpu.sync_copy(st_d.at[pl.ds(0, FCH + 16)],
                            db_hbm.at[pl.ds(off, FCH + 16)])
            return cursor + cnt_pad

        cursor = lax.fori_loop(0, NCHUNK, chunk_body, jnp.int32(0))
        for j in range(ECH // 16):
            st_s[pl.ds(j * 16, 16)] = jnp.zeros((16,), jnp.int32)
            st_d[pl.ds(j * 16, 16)] = jnp.full((16,), ROWS, jnp.int32)
        off = pl.multiple_of(wbase + cursor, 16)
        pltpu.sync_copy(st_s.at[pl.ds(0, ECH)], sb_hbm.at[pl.ds(off, ECH)])
        pltpu.sync_copy(st_d.at[pl.ds(0, ECH)], db_hbm.at[pl.ds(off, ECH)])
        nch = (cursor + ECH - 1) // ECH
        st_s[pl.ds(0, 16)] = jnp.full((16,), 1, jnp.int32) * nch
        pltpu.sync_copy(st_s.at[pl.ds(0, 16)],
                        cnt_hbm.at[pl.ds(pl.multiple_of(wid * 16, 16), 16)])

        nche = nch * ECH

        def rb_chunk(ck, curs):
            c0, c1 = curs
            roff = pl.multiple_of(wbase + ck * FCH2, 16)
            pltpu.sync_copy(sb_hbm.at[pl.ds(roff, FCH2)],
                            s_in.at[pl.ds(0, FCH2)])
            pltpu.sync_copy(db_hbm.at[pl.ds(roff, FCH2)],
                            d_in.at[pl.ds(0, FCH2)])

            def rb_vec(j, carry):
                l0, l1 = carry
                sv = s_in[pl.ds(j * 16, 16)]
                dl = d_in[pl.ds(j * 16, 16)]
                valid = (ck * FCH2 + j * 16 + lane) < nche
                m0 = valid & (dl < RSUB)
                m1 = valid & (dl >= RSUB) & (dl < ROWS)
                mi0 = m0.astype(jnp.int32)
                mi1 = m1.astype(jnp.int32)
                e0 = plsc.cumsum(mi0) - mi0
                e1 = plsc.cumsum(mi1) - mi1
                i0 = jnp.where(m0, l0 + e0, FCH2)
                i1 = jnp.where(m1, 4096 + l1 + e1, 4096 + FCH2)
                plsc.store_scatter(st_s, [i0], sv)
                plsc.store_scatter(st_d, [i0], dl)
                plsc.store_scatter(st_s, [i1], sv)
                plsc.store_scatter(st_d, [i1], dl - RSUB)
                return l0 + jnp.sum(mi0), l1 + jnp.sum(mi1)

            l0, l1 = lax.fori_loop(0, FCH2 // 16, rb_vec,
                                   (jnp.int32(0), jnp.int32(0)))
            st_s[pl.ds(l0, 16)] = jnp.zeros((16,), jnp.int32)
            st_d[pl.ds(l0, 16)] = jnp.full((16,), RSUB, jnp.int32)
            st_s[pl.ds(4096 + l1, 16)] = jnp.zeros((16,), jnp.int32)
            st_d[pl.ds(4096 + l1, 16)] = jnp.full((16,), RSUB, jnp.int32)
            o0 = pl.multiple_of(w2base + c0, 16)
            o1 = pl.multiple_of(w2base + CAP + c1, 16)
            pltpu.sync_copy(st_s.at[pl.ds(0, FCH2 + 16)],
                            sb2_hbm.at[pl.ds(o0, FCH2 + 16)])
            pltpu.sync_copy(st_d.at[pl.ds(0, FCH2 + 16)],
                            db2_hbm.at[pl.ds(o0, FCH2 + 16)])
            pltpu.sync_copy(st_s.at[pl.ds(4096, FCH2 + 16)],
                            sb2_hbm.at[pl.ds(o1, FCH2 + 16)])
            pltpu.sync_copy(st_d.at[pl.ds(4096, FCH2 + 16)],
                            db2_hbm.at[pl.ds(o1, FCH2 + 16)])
            return (c0 + ((l0 + 15) // 16) * 16, c1 + ((l1 + 15) // 16) * 16)

        nck2 = (nche + FCH2 - 1) // FCH2
        c0, c1 = lax.fori_loop(0, nck2, rb_chunk,
                               (jnp.int32(0), jnp.int32(0)))
        for j in range(ECHB // 16):
            st_s[pl.ds(j * 16, 16)] = jnp.zeros((16,), jnp.int32)
            st_d[pl.ds(j * 16, 16)] = jnp.full((16,), RSUB, jnp.int32)
        pltpu.sync_copy(st_s.at[pl.ds(0, ECHB)],
                        sb2_hbm.at[pl.ds(pl.multiple_of(w2base + c0, 16), ECHB)])
        pltpu.sync_copy(st_d.at[pl.ds(0, ECHB)],
                        db2_hbm.at[pl.ds(pl.multiple_of(w2base + c0, 16), ECHB)])
        pltpu.sync_copy(st_s.at[pl.ds(0, ECHB)],
                        sb2_hbm.at[pl.ds(pl.multiple_of(w2base + CAP + c1, 16), ECHB)])
        pltpu.sync_copy(st_d.at[pl.ds(0, ECHB)],
                        db2_hbm.at[pl.ds(pl.multiple_of(w2base + CAP + c1, 16), ECHB)])
        n0 = (c0 + ECHB - 1) // ECHB
        n1 = (c1 + ECHB - 1) // ECHB
        st_s[pl.ds(0, 16)] = jnp.where(lane < 1, n0, n1)
        pltpu.sync_copy(st_s.at[pl.ds(0, 16)],
                        cnt2_hbm.at[pl.ds(pl.multiple_of(wid * 16, 16), 16)])

    return filt(s, d)


def _gat_pass_sc(hs_list, es_T, ed_T, sb, db, cnt):
    nh = len(hs_list)
    mesh = plsc.VectorSubcoreMesh(core_axis_name="c", subcore_axis_name="s")

    @functools.partial(
        pl.kernel,
        out_type=(jax.ShapeDtypeStruct((nh * NPAD * C,), jnp.float32),
                  jax.ShapeDtypeStruct((nh * NW * DENW,), jnp.float32)),
        mesh=mesh,
        compiler_params=pltpu.CompilerParams(needs_layout_passes=False),
        scratch_types=[pltpu.VMEM(((ROWS + 1) * C,), jnp.float32),
                       pltpu.VMEM((ECH, C), jnp.float32),
                       pltpu.VMEM((ECH, C), jnp.float32),
                       pltpu.VMEM((N,), jnp.float32),
                       pltpu.VMEM((N,), jnp.float32),
                       pltpu.VMEM((DENW,), jnp.float32),
                       pltpu.VMEM((ECH,), jnp.int32),
                       pltpu.VMEM((ECH,), jnp.int32),
                       pltpu.VMEM((ECH,), jnp.int32),
                       pltpu.VMEM((ECH,), jnp.int32),
                       pltpu.VMEM((ECH,), jnp.float32),
                       pltpu.VMEM((16,), jnp.int32),
                       pltpu.SemaphoreType.DMA,
                       pltpu.SemaphoreType.DMA],
    )
    def gat(*refs):
        hs_refs = refs[:nh]
        es_hbm, ed_hbm, sb_hbm, db_hbm, cnt_hbm, raw_hbm, den_hbm = refs[nh:nh + 7]
        (out_f, rows0, rows1, es_v, ed_v, den_v, s_v0, s_v1, d_v0, d_v1,
         ex_v, cnt_v, sem0, sem1) = refs[nh + 7:]
        wid = lax.axis_index("s") * NC + lax.axis_index("c")
        lo = wid * ROWS
        wbase = wid * CAP
        pltpu.sync_copy(cnt_hbm.at[pl.ds(pl.multiple_of(wid * 16, 16), 16)],
                        cnt_v)
        nch = cnt_v[pl.ds(0, 16)][0]
        for h in range(nh):
            hs_ref = hs_refs[h]
            pltpu.sync_copy(es_hbm.at[pl.ds(h * N, N)], es_v)
            pltpu.sync_copy(ed_hbm.at[pl.ds(h * N, N)], ed_v)

            def zbody(i, _):
                out_f[pl.ds(i * 16, 16)] = jnp.zeros((16,), jnp.float32)
                return 0

            lax.fori_loop(0, (ROWS + 1) * C // 16, zbody, 0)
            for i in range(DENW // 16):
                den_v[pl.ds(i * 16, 16)] = jnp.zeros((16,), jnp.float32)

            def fetch(ck, s_v, d_v, rows, sem):
                @pl.when(ck < nch)
                def _():
                    eoff = pl.multiple_of(wbase + ck * ECH, 16)
                    pltpu.sync_copy(sb_hbm.at[pl.ds(eoff, ECH)], s_v)
                    pltpu.sync_copy(db_hbm.at[pl.ds(eoff, ECH)], d_v)
                    pltpu.make_async_copy(hs_ref.at[s_v], rows, sem).start()

            def process(ck, s_v, d_v, rows, sem):
                @pl.when(ck < nch)
                def _():
                    pltpu.make_async_copy(hs_ref.at[s_v], rows, sem).wait()

                    def vec_body(j, _):
                        sv = s_v[pl.ds(j * 16, 16)]
                        dv = d_v[pl.ds(j * 16, 16)]
                        esg = plsc.load_gather(es_v, [sv])
                        edi = jnp.minimum(dv + lo, N - 1)
                        edg = plsc.load_gather(ed_v, [edi])
                        e = esg + edg
                        e = jnp.where(e >= 0.0, e, 0.2 * e)
                        exv = jnp.exp(e)
                        plsc.addupdate_scatter(den_v, [dv], exv)
                        ex_v[pl.ds(j * 16, 16)] = exv
                        return 0

                    lax.fori_loop(0, ECH // 16, vec_body, 0)

                    def fma_body(k, _):
                        dv16 = d_v[pl.ds(k * 16, 16)]
                        ex16 = ex_v[pl.ds(k * 16, 16)]
                        for l in range(16):
                            base = dv16[l] * C
                            exb = jnp.full((16,), ex16[l])
                            for j in range(C // 16):
                                plsc.addupdate(
                                    out_f.at[pl.ds(base + j * 16, 16)],
                                    exb * rows[k * 16 + l, pl.ds(j * 16, 16)])
                        return 0

                    lax.fori_loop(0, ECH // 16, fma_body, 0)

            fetch(jnp.int32(0), s_v0, d_v0, rows0, sem0)

            def pair_body(k, _):
                c0 = k * 2
                fetch(c0 + 1, s_v1, d_v1, rows1, sem1)
                process(c0, s_v0, d_v0, rows0, sem0)
                fetch(c0 + 2, s_v0, d_v0, rows0, sem0)
                process(c0 + 1, s_v1, d_v1, rows1, sem1)
                return 0

            lax.fori_loop(0, (nch + 1) // 2, pair_body, 0)
            roff = pl.multiple_of(h * NPAD * C + lo * C, 16)
            pltpu.sync_copy(out_f.at[pl.ds(0, ROWS * C)],
                            raw_hbm.at[pl.ds(roff, ROWS * C)])
            doff = pl.multiple_of(h * NW * DENW + wid * DENW, 16)
            pltpu.sync_copy(den_v, den_hbm.at[pl.ds(doff, DENW)])

    raw_f, den_f = gat(*hs_list, es_T.reshape(-1), ed_T.reshape(-1), sb, db, cnt)
    raw = raw_f.reshape(nh, NPAD, C)
    den = den_f.reshape(nh, NW, DENW)[:, :, :ROWS].reshape(nh, NPAD)
    return raw, den


def _gat_passA_sc(es_i, ed_i, sb, db, cnt):
    mesh = plsc.VectorSubcoreMesh(core_axis_name="c", subcore_axis_name="s")

    @functools.partial(
        pl.kernel,
        out_type=(jax.ShapeDtypeStruct((NW * 2 * CAP * 2,), jnp.float32),
                  jax.ShapeDtypeStruct((NW * 2 * CAP * 2,), jnp.float32),
                  jax.ShapeDtypeStruct((NW * 4 * DENW,), jnp.float32)),
        mesh=mesh,
        compiler_params=pltpu.CompilerParams(needs_layout_passes=False),
        scratch_types=[pltpu.VMEM((4 * N,), jnp.float32),
                       pltpu.VMEM((4 * N,), jnp.float32),
                       pltpu.VMEM((ECHA,), jnp.int32),
                       pltpu.VMEM((ECHA,), jnp.int32),
                       pltpu.VMEM((2 * ECHA,), jnp.float32),
                       pltpu.VMEM((2 * ECHA,), jnp.float32),
                       pltpu.VMEM((4 * DENW,), jnp.float32),
                       pltpu.VMEM((16,), jnp.int32)],
    )
    def pA(es_hbm, ed_hbm, sb_hbm, db_hbm, cnt_hbm, ex0_hbm, ex1_hbm, den_hbm,
           es_v, ed_v, s_v, d_v, x0_v, x1_v, den_v, cnt_v):
        wid = lax.axis_index("s") * NC + lax.axis_index("c")
        lo = wid * ROWS
        pltpu.sync_copy(cnt_hbm.at[pl.ds(pl.multiple_of(wid * 16, 16), 16)],
                        cnt_v)
        pltpu.sync_copy(es_hbm, es_v)
        pltpu.sync_copy(ed_hbm, ed_v)
        for i in range(4 * DENW // 16):
            den_v[pl.ds(i * 16, 16)] = jnp.zeros((16,), jnp.float32)
        lane = lax.iota(jnp.int32, 16)
        cntv16 = cnt_v[pl.ds(0, 16)]

        def sub_body(b, _):
            base2 = wid * 2 * CAP + b * CAP
            nche = jnp.where(b == 0, cntv16[0], cntv16[1]) * ECHB
            ncha = (nche + ECHA - 1) // ECHA

            def chunk_body(ck, _):
                eoff = pl.multiple_of(base2 + ck * ECHA, 16)
                pltpu.sync_copy(sb_hbm.at[pl.ds(eoff, ECHA)], s_v)
                pltpu.sync_copy(db_hbm.at[pl.ds(eoff, ECHA)], d_v)

                def vec_body(j, _):
                    valid = (ck * ECHA + j * 16 + lane) < nche
                    sv = jnp.where(valid, s_v[pl.ds(j * 16, 16)], 0)
                    dsub = jnp.where(valid, d_v[pl.ds(j * 16, 16)], RSUB)
                    dv = jnp.where(dsub >= RSUB, ROWS, b * RSUB + dsub)
                    edi = jnp.minimum(dv + lo, N - 1) * 4
                    svi = sv * 4
                    for h in range(4):
                        esg = plsc.load_gather(es_v, [svi + h])
                        edg = plsc.load_gather(ed_v, [edi + h])
                        e = esg + edg
                        e = jnp.where(e >= 0.0, e, 0.2 * e)
                        exv = jnp.exp(e)
                        plsc.addupdate_scatter(den_v, [dv + h * DENW], exv)
                        xst = x0_v if h < 2 else x1_v
                        plsc.store_scatter(xst,
                                           [lane * 2 + (j * 32 + (h % 2))],
                                           exv)
                    return 0

                lax.fori_loop(0, ECHA // 16, vec_body, 0)
                xoff = pl.multiple_of(2 * (base2 + ck * ECHA), 16)
                pltpu.sync_copy(x0_v, ex0_hbm.at[pl.ds(xoff, 2 * ECHA)])
                pltpu.sync_copy(x1_v, ex1_hbm.at[pl.ds(xoff, 2 * ECHA)])
                return 0

            lax.fori_loop(0, ncha, chunk_body, 0)
            return 0

        lax.fori_loop(0, 2, sub_body, 0)
        doff = pl.multiple_of(wid * 4 * DENW, 16)
        pltpu.sync_copy(den_v, den_hbm.at[pl.ds(doff, 4 * DENW)])

    ex0, ex1, den_f = pA(es_i, ed_i, sb, db, cnt)
    den = (den_f.reshape(NW, 4, DENW)[:, :, :ROWS]
           .transpose(1, 0, 2).reshape(4, NPAD))
    return ex0, ex1, den


def _gat_passB_sc(hs_pairs, sb, db, cnt, ex0, ex1):
    mesh = plsc.VectorSubcoreMesh(core_axis_name="c", subcore_axis_name="s")
    C2 = 2 * C

    @functools.partial(
        pl.kernel,
        out_type=jax.ShapeDtypeStruct((2 * NPAD * C2,), jnp.float32),
        mesh=mesh,
        compiler_params=pltpu.CompilerParams(needs_layout_passes=False),
        scratch_types=[pltpu.VMEM(((RSUB + 1) * C2,), jnp.float32),
                       pltpu.VMEM((ECHB, C2), jnp.float32),
                       pltpu.VMEM((ECHB, C2), jnp.float32),
                       pltpu.VMEM((SCH * ECHB,), jnp.int32),
                       pltpu.VMEM((SCH * ECHB,), jnp.int32),
                       pltpu.VMEM((2 * SCH * ECHB,), jnp.float32),
                       pltpu.VMEM((16,), jnp.int32),
                       pltpu.SemaphoreType.DMA,
                       pltpu.SemaphoreType.DMA],
    )
    def pB(hs0_hbm, hs1_hbm, sb_hbm, db_hbm, cnt_hbm, ex0_hbm, ex1_hbm,
           raw_hbm, out_f, rows0, rows1, s_big, d_big, x_big,
           cnt_v, sem0, sem1):
        wid = lax.axis_index("s") * NC + lax.axis_index("c")
        lo = wid * ROWS
        pltpu.sync_copy(cnt_hbm.at[pl.ds(pl.multiple_of(wid * 16, 16), 16)],
                        cnt_v)
        cntv16 = cnt_v[pl.ds(0, 16)]
        for p in range(2):
            hs_ref = (hs0_hbm, hs1_hbm)[p]
            ex_ref = (ex0_hbm, ex1_hbm)[p]

            def sub_body(b, _):
                base2 = wid * 2 * CAP + b * CAP
                nchb = jnp.where(b == 0, cntv16[0], cntv16[1])

                def zbody(i, _):
                    out_f[pl.ds(i * 16, 16)] = jnp.zeros((16,), jnp.float32)
                    return 0

                lax.fori_loop(0, (RSUB + 1) * C2 // 16, zbody, 0)

                def idx_ref(j):
                    return s_big.at[pl.ds(pl.multiple_of(j * ECHB, 8), ECHB)]

                def fetch(cb, j, rows, sem):
                    @pl.when((j < SCH) & (cb + j < nchb))
                    def _():
                        pltpu.make_async_copy(hs_ref.at[idx_ref(j)], rows,
                                              sem).start()

                def process(cb, j, rows, sem):
                    @pl.when(cb + j < nchb)
                    def _():
                        pltpu.make_async_copy(hs_ref.at[idx_ref(j)], rows,
                                              sem).wait()

                        def fma_body(k, _):
                            kk = j * (ECHB // 16) + k
                            dv16 = d_big[pl.ds(kk * 16, 16)]
                            xa = x_big[pl.ds(kk * 32, 16)]
                            xb = x_big[pl.ds(kk * 32 + 16, 16)]
                            for l in range(16):
                                e = k * 16 + l
                                xv = xa if l < 8 else xb
                                base = dv16[l] * C2
                                for hh in range(2):
                                    exb = jnp.full((16,),
                                                   xv[(l % 8) * 2 + hh])
                                    for jj in range(C // 16):
                                        off = hh * C + jj * 16
                                        plsc.addupdate(
                                            out_f.at[pl.ds(base + off, 16)],
                                            exb * rows[e, pl.ds(off, 16)])
                            return 0

                        lax.fori_loop(0, ECHB // 16, fma_body, 0)

                def sup_body(sk, _):
                    cb = sk * SCH
                    soff = pl.multiple_of(base2 + cb * ECHB, 16)
                    pltpu.sync_copy(sb_hbm.at[pl.ds(soff, SCH * ECHB)],
                                    s_big)
                    pltpu.sync_copy(db_hbm.at[pl.ds(soff, SCH * ECHB)],
                                    d_big)
                    xoff = pl.multiple_of(2 * (base2 + cb * ECHB), 16)
                    pltpu.sync_copy(ex_ref.at[pl.ds(xoff, 2 * SCH * ECHB)],
                                    x_big)
                    fetch(cb, jnp.int32(0), rows0, sem0)

                    def pair_body(t, _):
                        j0 = t * 2
                        fetch(cb, j0 + 1, rows1, sem1)
                        process(cb, j0, rows0, sem0)
                        fetch(cb, j0 + 2, rows0, sem0)
                        process(cb, j0 + 1, rows1, sem1)
                        return 0

                    lax.fori_loop(0, SCH // 2, pair_body, 0)
                    return 0

                lax.fori_loop(0, (nchb + SCH - 1) // SCH, sup_body, 0)
                roff = pl.multiple_of(
                    p * NPAD * C2 + (lo + b * RSUB) * C2, 16)
                pltpu.sync_copy(out_f.at[pl.ds(0, RSUB * C2)],
                                raw_hbm.at[pl.ds(roff, RSUB * C2)])
                return 0

            lax.fori_loop(0, 2, sub_body, 0)

    raw_f = pB(hs_pairs[0], hs_pairs[1], sb, db, cnt, ex0, ex1)
    return raw_f.reshape(2, NPAD, C2)


def _blockdiag_a(a):
    hh = a.shape[0]
    eye = jnp.eye(hh, dtype=a.dtype)
    A = eye[:, :, None] * a[:, None, :]
    return A.transpose(1, 2, 0).reshape(hh * a.shape[1], hh)


def kernel(x_person, x_diagnosis, edge_index_p2d, edge_index_d2p, edge_attr_p2d,
           g1p_Ws, g1p_Wd, g1p_as, g1p_ad, g1p_b,
           g1d_Ws, g1d_Wd, g1d_as, g1d_ad, g1d_b,
           g2p_W, g2p_as, g2p_ad, g2p_b,
           g2d_W, g2d_as, g2d_ad, g2d_b,
           lin1_W, lin1_b, lin2_W, lin2_b, lin3_W, lin3_b,
           em1_W, em1_b, em2_W, em2_b):
    Asp = _blockdiag_a(g1p_as)
    Adp = _blockdiag_a(g1p_ad)
    Asd = _blockdiag_a(g1d_as)
    Add = _blockdiag_a(g1d_ad)
    vdp = g1p_Wd @ Adp
    vdd = g1d_Wd @ Add
    as2 = g2d_as.reshape(CH, 1)
    vd2 = g2d_W @ g2d_ad.reshape(CH, 1)

    hs_p2d, es_p2d, ed_p2d, hs_d2p, es_d2p, ed_d2p = _stage1(
        x_person, x_diagnosis, g1p_Ws, g1d_Ws, Asp, Asd, vdp, vdd)

    s_p2d = edge_index_p2d[0]
    d_p2d = edge_index_p2d[1]
    s_d2p = edge_index_d2p[0]
    d_d2p = edge_index_d2p[1]

    sb_p, db_p, cnt_p, sb2_p, db2_p, cnt2_p = _filter_sc(s_p2d, d_p2d)
    sb_d, db_d, cnt_d, sb2_d, db2_d, cnt2_d = _filter_sc(s_d2p, d_d2p)

    ex0_p, ex1_p, den_d1 = _gat_passA_sc(
        es_p2d.reshape(-1), ed_p2d.reshape(-1), sb2_p, db2_p, cnt2_p)
    raw_d1 = _gat_passB_sc(
        [hs_p2d[0], hs_p2d[1]], sb2_p, db2_p, cnt2_p, ex0_p, ex1_p)
    ex0_d, ex1_d, den_p1 = _gat_passA_sc(
        es_d2p.reshape(-1), ed_d2p.reshape(-1), sb2_d, db2_d, cnt2_d)
    raw_p1 = _gat_passB_sc(
        [hs_d2p[0], hs_d2p[1]], sb2_d, db2_d, cnt2_d, ex0_d, ex1_d)

    hs2, es2 = _stage5d(raw_d1, den_d1, g1p_b, lin1_W, lin1_b, g2d_W, as2)
    ed2 = _stage5p(raw_p1, den_p1, g1d_b, lin1_W, lin1_b, vd2)

    raw2, den2 = _gat_pass_sc([hs2], es2.T, ed2.T, sb_d, db_d, cnt_d)

    return _stage8(raw2[0], den2.reshape(NPAD, 1),
                   g2d_b, lin2_W, lin2_b, lin3_W, lin3_b)
